# Initial kernel scaffold; baseline (speedup 1.0000x reference)
import jax
import jax.numpy as jnp
from jax.experimental import pallas as pl


def kernel(x, emb_table, pos_table):
    raise NotImplementedError("write your pallas kernel here")


# speedup vs baseline: 6.7451x; 6.7451x over previous
import functools

import jax
import jax.numpy as jnp
from jax import lax
from jax.experimental import pallas as pl
from jax.experimental.pallas import tpu as pltpu
from jax.experimental.pallas import tpu_sc as plsc

B, S, D, V = 4096, 200, 128, 1000
N = B * S
NC, NS, L = 2, 16, 16
NW = NC * NS
PER_W = N // NW
CHUNK = S
NCHUNK = PER_W // CHUNK
NBUF = 3
IA = 128
IB = CHUNK - IA

_mesh = plsc.VectorSubcoreMesh(
    core_axis_name="c", subcore_axis_name="s", num_cores=NC, num_subcores=NS
)

_scratch = (
    [pltpu.VMEM((PER_W,), jnp.int32)]
    + [pltpu.VMEM((S, D), jnp.float32)]
    + [pltpu.VMEM((CHUNK, D), jnp.float32)] * NBUF
    + [pltpu.SemaphoreType.DMA] * (2 * NBUF)
)


@functools.partial(
    pl.kernel,
    out_type=jax.ShapeDtypeStruct((N, D), jnp.float32),
    mesh=_mesh,
    scratch_types=_scratch,
)
def _embed_sc(x_hbm, emb_hbm, pos_hbm, out_hbm,
              idx_v, pos_v, r0, r1, r2, g0, g1, g2, s0, s1, s2):
    rows = (r0, r1, r2)
    gsem = (g0, g1, g2)
    ssem = (s0, s1, s2)

    wid = lax.axis_index("s") * NC + lax.axis_index("c")
    base = wid * PER_W

    pltpu.sync_copy(x_hbm.at[pl.ds(base, PER_W)], idx_v)
    pltpu.sync_copy(pos_hbm.at[pl.ds(0, S)], pos_v)

    def gather_descs(k, b):
        off = pl.multiple_of(k * CHUNK, 8)
        da = pltpu.make_async_copy(
            emb_hbm.at[idx_v.at[pl.ds(off, IA)]],
            rows[b].at[pl.ds(0, IA)], gsem[b])
        db = pltpu.make_async_copy(
            emb_hbm.at[idx_v.at[pl.ds(off + IA, IB)]],
            rows[b].at[pl.ds(IA, IB)], gsem[b])
        return da, db

    def start_gather(k, b):
        da, db = gather_descs(k, b)
        da.start()
        db.start()

    def wait_gather(k, b):
        da, db = gather_descs(k, b)
        da.wait()
        db.wait()

    def scatter_desc(k, b):
        off = pl.multiple_of(base + k * CHUNK, 8)
        return pltpu.make_async_copy(
            rows[b], out_hbm.at[pl.ds(off, CHUNK)], ssem[b])

    def add_pos(b):
        r = rows[b]

        def sbody(i, carry):
            for u in range(2):
                srow = i * 2 + u
                for d in range(D // L):
                    sl = pl.ds(d * L, L)
                    plsc.addupdate(r.at[srow, sl], pos_v[srow, sl])
            return carry

        lax.fori_loop(0, S // 2, sbody, 0)

    def process(k, b):
        wait_gather(k, b)
        add_pos(b)
        scatter_desc(k, b).start()

    start_gather(0, 0)
    for k in range(NBUF):
        k1 = k + 1
        b1 = k1 % NBUF
        if k1 >= NBUF:
            scatter_desc(k1 - NBUF, b1).wait()
        start_gather(k1, b1)
        process(k, k % NBUF)

    def group(g, carry):
        for b in range(NBUF):
            k = g * NBUF + b
            k1 = k + 1
            b1 = (b + 1) % NBUF
            scatter_desc(k1 - NBUF, b1).wait()
            start_gather(k1, b1)
            process(k, b)
        return carry

    groups_end = (NCHUNK - 2) // NBUF
    lax.fori_loop(1, groups_end, group, 0)

    k = NCHUNK - 2
    scatter_desc(k + 1 - NBUF, (k + 1) % NBUF).wait()
    start_gather(k + 1, (k + 1) % NBUF)
    process(k, k % NBUF)
    process(k + 1, (k + 1) % NBUF)
    for j in range(NCHUNK - NBUF, NCHUNK):
        scatter_desc(j, j % NBUF).wait()


def kernel(x, emb_table, pos_table):
    flat = x.astype(jnp.int32).reshape(N)
    out = _embed_sc(flat, emb_table, pos_table)
    return out.reshape(B, S, D)
